# baseline (speedup 1.0000x reference)
import jax
import jax.numpy as jnp
from jax.experimental import pallas as pl
from jax.experimental.pallas import tpu as pltpu

F32 = jnp.float32
BF16 = jnp.bfloat16
BT = 16
P1 = 76
P2 = 38
P3 = 19
NR = BT * P1
M1 = NR - 2
NP1 = BT * P2
M2 = NP1 - 2
NP2 = BT * P3 - 1


def _shift_lanes_left(a):
    return jnp.concatenate([a[:, 1:], a[:, :1]], axis=1)


def _fused_kernel(x_ref,
                  w1_ref,
                  w2_ref,
                  m1_ref,
                  seg_ref,
                  b1_ref,
                  b2_ref,
                  dmask_ref,
                  fold_ref,
                  fb1_ref, fw2_ref, fb2_ref,
                  o_ref,
                  slab_ref,
                  wm1_ref,
                  p1_ref,
                  wm2_ref):
    for bi in range(BT):
        for ic in range(12):
            slab_ref[bi * P1:bi * P1 + 74, ic * 98:(ic + 1) * 98] = (
                x_ref[bi, ic].astype(BF16))
        slab_ref[bi * P1 + 74:(bi + 1) * P1, :] = jnp.zeros((2, 1176), BF16)

    acc = jnp.dot(slab_ref[0:M1, :], w1_ref[0], preferred_element_type=F32)
    acc = acc + jnp.dot(slab_ref[1:M1 + 1, :], w1_ref[1],
                        preferred_element_type=F32)
    acc = acc + jnp.dot(slab_ref[2:M1 + 2, :], w1_ref[2],
                        preferred_element_type=F32)
    act = jnp.maximum(acc + b1_ref[...], 0.0)
    wmax1 = jnp.maximum(act, _shift_lanes_left(act))
    for j in range(6):
        wm1_ref[j, 0:M1, :] = wmax1[:, j * 128:(j + 1) * 128]

    for j in range(6):
        p1_ref[0:NP1 - 1, j * 128:(j + 1) * 128] = jnp.maximum(
            wm1_ref[j, 0:M1:2, :], wm1_ref[j, 1:M1:2, :]).astype(BF16)
    p1_ref[NP1 - 1:NP1, :] = jnp.zeros((1, 768), BF16)

    acc2 = jnp.dot(p1_ref[0:M2, :], w2_ref[0], preferred_element_type=F32)
    acc2 = acc2 + jnp.dot(p1_ref[1:M2 + 1, :], w2_ref[1],
                          preferred_element_type=F32)
    acc2 = acc2 + jnp.dot(p1_ref[2:M2 + 2, :], w2_ref[2],
                          preferred_element_type=F32)
    act2 = jnp.maximum(acc2 + b2_ref[...], 0.0)
    wmax2 = jnp.maximum(act2, _shift_lanes_left(act2))
    for j in range(2):
        wm2_ref[j, 0:M2, :] = wmax2[:, j * 128:(j + 1) * 128]

    p2 = jnp.concatenate(
        [jnp.maximum(wm2_ref[j, 0:M2:2, :], wm2_ref[j, 1:M2:2, :])
         for j in range(2)], axis=1)
    g = jnp.dot(p2.astype(BF16), m1_ref[...],
                preferred_element_type=F32)
    masked = (g * dmask_ref[...]).astype(BF16)
    f = jnp.dot(seg_ref[...], masked, preferred_element_type=F32)

    h = jnp.maximum(jnp.dot(f.astype(BF16), fold_ref[...],
                            preferred_element_type=F32) + fb1_ref[...], 0.0)
    o_ref[0] = jnp.dot(h.astype(BF16), fw2_ref[...],
                       preferred_element_type=F32) + fb2_ref[...]


def kernel(x, conv1_w_flat, conv1_b, conv2_b, w2_lane, csel,
           pool_swe, pool_swo, pool_she, pool_sho,
           fc1_m, dmask, fold, fc1_b, fc2_wt, fc2_b):
    B = x.shape[0]

    cw1 = conv1_w_flat.reshape(8, 12, 3, 3)
    wp = jnp.arange(98)[:, None]
    w = jnp.arange(96)[None, :]
    s1 = jnp.stack([(wp == w + kw).astype(F32) for kw in range(3)])
    w1t = jnp.einsum('kpw,oihk->hipow', s1, cw1).reshape(3, 1176, 768)

    cw2 = w2_lane.reshape(4, 9, 8, 48)[:, :, :, 0].reshape(4, 3, 3, 8)
    cw2 = cw2.transpose(0, 3, 1, 2)
    u = jnp.arange(96)[:, None]
    w2c = jnp.arange(46)[None, :]
    s2 = jnp.stack([(u == 2 * (w2c + kw)).astype(F32) for kw in range(3)])
    w2t = jnp.einsum('kuw,oihk->hiuow', s2, cw2).reshape(3, 768, 184)
    w2t = jnp.concatenate([w2t, jnp.zeros((3, 768, 72), F32)], axis=2)

    m1p = jnp.zeros((4, 46, 544), F32).at[:, 0::2, :].set(fc1_m)
    m1p = jnp.concatenate([m1p.reshape(184, 544),
                           jnp.zeros((72, 544), F32)], axis=0)

    dmask19 = jnp.zeros((BT, P3, 544), F32).at[:, :17, :].set(
        jnp.broadcast_to(dmask, (BT, 17, 544)))
    dmask19 = dmask19.reshape(BT * P3, 544)[0:NP2]
    seg = (jnp.arange(NP2)[None, :] // P3
           == jnp.arange(BT)[:, None]).astype(BF16)

    b1row = jnp.repeat(conv1_b, 96).reshape(1, 768)
    b2row = jnp.concatenate([jnp.repeat(conv2_b, 46),
                             jnp.zeros((72,), F32)]).reshape(1, 256)

    grid = (B // BT,)
    out = pl.pallas_call(
        _fused_kernel,
        out_shape=jax.ShapeDtypeStruct((B // BT, BT, 24), F32),
        grid=grid,
        in_specs=[
            pl.BlockSpec((BT, 12, 74, 98), lambda b: (b, 0, 0, 0)),
            pl.BlockSpec((3, 1176, 768), lambda b: (0, 0, 0)),
            pl.BlockSpec((3, 768, 256), lambda b: (0, 0, 0)),
            pl.BlockSpec((256, 544), lambda b: (0, 0)),
            pl.BlockSpec((BT, NP2), lambda b: (0, 0)),
            pl.BlockSpec((1, 768), lambda b: (0, 0)),
            pl.BlockSpec((1, 256), lambda b: (0, 0)),
            pl.BlockSpec((NP2, 544), lambda b: (0, 0)),
            pl.BlockSpec((544, 32), lambda b: (0, 0)),
            pl.BlockSpec((1, 32), lambda b: (0, 0)),
            pl.BlockSpec((32, 24), lambda b: (0, 0)),
            pl.BlockSpec((1, 24), lambda b: (0, 0)),
        ],
        out_specs=pl.BlockSpec((1, BT, 24), lambda b: (b, 0, 0)),
        scratch_shapes=[
            pltpu.VMEM((NR, 1176), BF16),
            pltpu.VMEM((6, NR, 128), F32),
            pltpu.VMEM((NP1, 768), BF16),
            pltpu.VMEM((2, NP1, 128), F32),
        ],
        compiler_params=pltpu.CompilerParams(
            dimension_semantics=("parallel",),
            vmem_limit_bytes=100 * 1024 * 1024),
    )(x, w1t.astype(BF16), w2t.astype(BF16), m1p.astype(BF16),
      seg, b1row, b2row, dmask19,
      fold.astype(BF16), fc1_b, fc2_wt.astype(BF16), fc2_b)
    return out.reshape(B, 24)


# speedup vs baseline: 1.0225x; 1.0225x over previous
import jax
import jax.numpy as jnp
from jax.experimental import pallas as pl
from jax.experimental.pallas import tpu as pltpu

F32 = jnp.float32
BF16 = jnp.bfloat16
BT = 16
P1 = 76
P2 = 38
P3 = 19
NR = BT * P1
M1 = NR - 2
NP1 = BT * P2
M2 = NP1 - 2
NP2 = BT * P3 - 1


def _shift_lanes_left(a):
    return jnp.concatenate([a[:, 1:], a[:, :1]], axis=1)


def _fused_kernel(x_ref,
                  w1_ref,
                  w2_ref,
                  m1_ref,
                  seg_ref,
                  b1_ref,
                  b2_ref,
                  dmask_ref,
                  fold_ref,
                  fb1_ref, fw2_ref, fb2_ref,
                  o_ref,
                  slab_ref,
                  wm1_ref,
                  p1_ref,
                  wm2_ref):
    for bi in range(BT):
        for ic in range(12):
            slab_ref[bi * P1:bi * P1 + 74, ic * 98:(ic + 1) * 98] = (
                x_ref[bi, ic].astype(BF16))
        slab_ref[bi * P1 + 74:(bi + 1) * P1, :] = jnp.zeros((2, 1176), BF16)

    acc = jnp.dot(slab_ref[0:M1, :], w1_ref[0], preferred_element_type=F32)
    acc = acc + jnp.dot(slab_ref[1:M1 + 1, :], w1_ref[1],
                        preferred_element_type=F32)
    acc = acc + jnp.dot(slab_ref[2:M1 + 2, :], w1_ref[2],
                        preferred_element_type=F32)
    for j in range(6):
        wm1_ref[j, 0:M1, :] = acc[:, j * 128:(j + 1) * 128]

    for j in range(6):
        rm = jnp.maximum(wm1_ref[j, 0:M1:2, :], wm1_ref[j, 1:M1:2, :])
        wm = jnp.maximum(rm, _shift_lanes_left(rm))
        p1_ref[0:NP1 - 1, j * 128:(j + 1) * 128] = jnp.maximum(
            wm + b1_ref[:, j * 128:(j + 1) * 128], 0.0).astype(BF16)
    p1_ref[NP1 - 1:NP1, :] = jnp.zeros((1, 768), BF16)

    acc2 = jnp.dot(p1_ref[0:M2, :], w2_ref[0], preferred_element_type=F32)
    acc2 = acc2 + jnp.dot(p1_ref[1:M2 + 1, :], w2_ref[1],
                          preferred_element_type=F32)
    acc2 = acc2 + jnp.dot(p1_ref[2:M2 + 2, :], w2_ref[2],
                          preferred_element_type=F32)
    for j in range(2):
        wm2_ref[j, 0:M2, :] = acc2[:, j * 128:(j + 1) * 128]

    p2blks = []
    for j in range(2):
        rm2 = jnp.maximum(wm2_ref[j, 0:M2:2, :], wm2_ref[j, 1:M2:2, :])
        wm2 = jnp.maximum(rm2, _shift_lanes_left(rm2))
        p2blks.append(jnp.maximum(
            wm2 + b2_ref[:, j * 128:(j + 1) * 128], 0.0).astype(BF16))
    p2 = jnp.concatenate(p2blks, axis=1)
    g = jnp.dot(p2, m1_ref[...],
                preferred_element_type=F32)
    masked = (g * dmask_ref[...]).astype(BF16)
    f = jnp.dot(seg_ref[...], masked, preferred_element_type=F32)

    h = jnp.maximum(jnp.dot(f.astype(BF16), fold_ref[...],
                            preferred_element_type=F32) + fb1_ref[...], 0.0)
    o_ref[0] = jnp.dot(h.astype(BF16), fw2_ref[...],
                       preferred_element_type=F32) + fb2_ref[...]


def kernel(x, conv1_w_flat, conv1_b, conv2_b, w2_lane, csel,
           pool_swe, pool_swo, pool_she, pool_sho,
           fc1_m, dmask, fold, fc1_b, fc2_wt, fc2_b):
    B = x.shape[0]

    cw1 = conv1_w_flat.reshape(8, 12, 3, 3)
    wp = jnp.arange(98)[:, None]
    w = jnp.arange(96)[None, :]
    s1 = jnp.stack([(wp == w + kw).astype(F32) for kw in range(3)])
    w1t = jnp.einsum('kpw,oihk->hipow', s1, cw1).reshape(3, 1176, 768)

    cw2 = w2_lane.reshape(4, 9, 8, 48)[:, :, :, 0].reshape(4, 3, 3, 8)
    cw2 = cw2.transpose(0, 3, 1, 2)
    u = jnp.arange(96)[:, None]
    w2c = jnp.arange(46)[None, :]
    s2 = jnp.stack([(u == 2 * (w2c + kw)).astype(F32) for kw in range(3)])
    w2t = jnp.einsum('kuw,oihk->hiuow', s2, cw2).reshape(3, 768, 184)
    w2t = jnp.concatenate([w2t, jnp.zeros((3, 768, 72), F32)], axis=2)

    m1p = jnp.zeros((4, 46, 544), F32).at[:, 0::2, :].set(fc1_m)
    m1p = jnp.concatenate([m1p.reshape(184, 544),
                           jnp.zeros((72, 544), F32)], axis=0)

    dmask19 = jnp.zeros((BT, P3, 544), F32).at[:, :17, :].set(
        jnp.broadcast_to(dmask, (BT, 17, 544)))
    dmask19 = dmask19.reshape(BT * P3, 544)[0:NP2]
    seg = (jnp.arange(NP2)[None, :] // P3
           == jnp.arange(BT)[:, None]).astype(BF16)

    b1row = jnp.repeat(conv1_b, 96).reshape(1, 768)
    b2row = jnp.concatenate([jnp.repeat(conv2_b, 46),
                             jnp.zeros((72,), F32)]).reshape(1, 256)

    grid = (B // BT,)
    out = pl.pallas_call(
        _fused_kernel,
        out_shape=jax.ShapeDtypeStruct((B // BT, BT, 24), F32),
        grid=grid,
        in_specs=[
            pl.BlockSpec((BT, 12, 74, 98), lambda b: (b, 0, 0, 0)),
            pl.BlockSpec((3, 1176, 768), lambda b: (0, 0, 0)),
            pl.BlockSpec((3, 768, 256), lambda b: (0, 0, 0)),
            pl.BlockSpec((256, 544), lambda b: (0, 0)),
            pl.BlockSpec((BT, NP2), lambda b: (0, 0)),
            pl.BlockSpec((1, 768), lambda b: (0, 0)),
            pl.BlockSpec((1, 256), lambda b: (0, 0)),
            pl.BlockSpec((NP2, 544), lambda b: (0, 0)),
            pl.BlockSpec((544, 32), lambda b: (0, 0)),
            pl.BlockSpec((1, 32), lambda b: (0, 0)),
            pl.BlockSpec((32, 24), lambda b: (0, 0)),
            pl.BlockSpec((1, 24), lambda b: (0, 0)),
        ],
        out_specs=pl.BlockSpec((1, BT, 24), lambda b: (b, 0, 0)),
        scratch_shapes=[
            pltpu.VMEM((NR, 1176), BF16),
            pltpu.VMEM((6, NR, 128), F32),
            pltpu.VMEM((NP1, 768), BF16),
            pltpu.VMEM((2, NP1, 128), F32),
        ],
        compiler_params=pltpu.CompilerParams(
            dimension_semantics=("parallel",),
            vmem_limit_bytes=100 * 1024 * 1024),
    )(x, w1t.astype(BF16), w2t.astype(BF16), m1p.astype(BF16),
      seg, b1row, b2row, dmask19,
      fold.astype(BF16), fc1_b, fc2_wt.astype(BF16), fc2_b)
    return out.reshape(B, 24)
